# baseline (speedup 1.0000x reference)
import functools

import jax
import jax.numpy as jnp
from jax import lax
from jax.experimental import pallas as pl
from jax.experimental.pallas import tpu as pltpu
from jax.experimental.pallas import tpu_sc as plsc

D = 128
L = 50
LPAD = 56
SEG = 2026
SEGPAD = 2048
NW = 32
SPW = SEGPAD // NW
VECS = D // 16


def _pool_body(emb_hbm, idx_hbm, out_hbm, idx_v, rows_v, out_v, sem):
    c = lax.axis_index("c")
    s = lax.axis_index("s")
    wid = s * 2 + c
    base = wid * SPW

    pltpu.sync_copy(idx_hbm.at[pl.ds(base, SPW)], idx_v)

    def seg_body(j, carry):
        pltpu.async_copy(emb_hbm.at[idx_v.at[j]], rows_v, sem).wait()

        def row_body(r, acc):
            return tuple(acc[k] + rows_v[r, pl.ds(16 * k, 16)]
                         for k in range(VECS))

        acc0 = tuple(jnp.zeros((16,), jnp.float32) for _ in range(VECS))
        acc = lax.fori_loop(0, L, row_body, acc0)
        for k in range(VECS):
            out_v[j, pl.ds(16 * k, 16)] = acc[k] * (1.0 / L)
        return carry

    lax.fori_loop(0, SPW, seg_body, 0)
    pltpu.sync_copy(out_v, out_hbm.at[pl.ds(base, SPW)])


@functools.partial(
    pl.kernel,
    out_type=jax.ShapeDtypeStruct((SEGPAD, D), jnp.float32),
    mesh=plsc.VectorSubcoreMesh(core_axis_name="c", subcore_axis_name="s"),
    scratch_types=[
        pltpu.VMEM((SPW, LPAD), jnp.int32),
        pltpu.VMEM((LPAD, D), jnp.float32),
        pltpu.VMEM((SPW, D), jnp.float32),
        pltpu.SemaphoreType.DMA,
    ],
)
def _pool_sc(emb_hbm, idx_hbm, out_hbm, idx_v, rows_v, out_v, sem):
    _pool_body(emb_hbm, idx_hbm, out_hbm, idx_v, rows_v, out_v, sem)


def _softmax0(x):
    m = jnp.max(x, axis=0, keepdims=True)
    e = jnp.exp(x - m)
    return e / jnp.sum(e, axis=0, keepdims=True)


def _dense_tc(pooled_ref, rw_ref, out_ref):
    pooled = pooled_ref[...]
    rw = rw_ref[...]
    enc_keys = pooled[0:1000]
    enc_values = pooled[1000:2000]
    enc_cands = pooled[2000:2020]
    enc_persona = pooled[2020:2025]
    enc_x = pooled[2025:2026]

    eps = 1e-6
    dot = jnp.sum(enc_x * enc_persona, axis=1, keepdims=True)
    na = jnp.sqrt(jnp.sum(enc_x * enc_x, axis=1, keepdims=True))
    nb = jnp.sqrt(jnp.sum(enc_persona * enc_persona, axis=1, keepdims=True))
    sim = dot / (jnp.maximum(na, eps) * jnp.maximum(nb, eps))
    ss = _softmax0(sim)
    test = jnp.dot(ss.T, enc_persona, preferred_element_type=jnp.float32)
    q = jnp.dot(test, rw.T, preferred_element_type=jnp.float32)
    tmp = jnp.dot(enc_keys, q.T, preferred_element_type=jnp.float32)
    ph = _softmax0(tmp)
    test2 = jnp.dot(ph.T, enc_values, preferred_element_type=jnp.float32)
    q2 = jnp.dot(test2, rw.T, preferred_element_type=jnp.float32)
    logits = jnp.dot(enc_cands, q2.T, preferred_element_type=jnp.float32)
    out_ref[...] = _softmax0(logits)


def kernel(xs, candidates, persona, label, keys, values, emb_table, R_W):
    del label
    idx = jnp.concatenate([
        keys.reshape(-1), values.reshape(-1), candidates.reshape(-1),
        persona.reshape(-1), xs.reshape(-1),
    ]).astype(jnp.int32).reshape(SEG, L)
    idx_pad = jnp.zeros((SEGPAD, LPAD), jnp.int32).at[:SEG, :L].set(idx)
    pooled = _pool_sc(emb_table.astype(jnp.float32), idx_pad)
    preds = pl.pallas_call(
        _dense_tc,
        out_shape=jax.ShapeDtypeStruct((20, 1), jnp.float32),
    )(pooled, R_W.astype(jnp.float32))
    return preds


# speedup vs baseline: 1.0174x; 1.0174x over previous
import functools

import jax
import jax.numpy as jnp
from jax import lax
from jax.experimental import pallas as pl
from jax.experimental.pallas import tpu as pltpu
from jax.experimental.pallas import tpu_sc as plsc

D = 128
L = 50
LPAD = 56
SEG = 2026
SEGPAD = 2048
NW = 32
SPW = SEGPAD // NW
VECS = D // 16


def _pool_body(emb_hbm, idx_hbm, out_hbm, idx_v, rows0, rows1, out_v,
               sem0, sem1):
    c = lax.axis_index("c")
    s = lax.axis_index("s")
    wid = s * 2 + c
    base = wid * SPW

    pltpu.sync_copy(idx_hbm.at[pl.ds(base, SPW)], idx_v)

    def accum(j, buf):
        def row_body(r, acc):
            return tuple(acc[k] + buf[r, pl.ds(16 * k, 16)]
                         for k in range(VECS))

        acc0 = tuple(jnp.zeros((16,), jnp.float32) for _ in range(VECS))
        acc = lax.fori_loop(0, L, row_body, acc0)
        for k in range(VECS):
            out_v[j, pl.ds(16 * k, 16)] = acc[k] * (1.0 / L)

    pltpu.async_copy(emb_hbm.at[idx_v.at[0]], rows0, sem0)

    def pair_body(t, carry):
        j0 = 2 * t
        pltpu.async_copy(emb_hbm.at[idx_v.at[j0 + 1]], rows1, sem1)
        pltpu.make_async_copy(emb_hbm.at[idx_v.at[j0]], rows0, sem0).wait()
        accum(j0, rows0)

        @pl.when(t < SPW // 2 - 1)
        def _():
            pltpu.async_copy(emb_hbm.at[idx_v.at[j0 + 2]], rows0, sem0)

        pltpu.make_async_copy(emb_hbm.at[idx_v.at[j0 + 1]], rows1, sem1).wait()
        accum(j0 + 1, rows1)
        return carry

    lax.fori_loop(0, SPW // 2, pair_body, 0)
    pltpu.sync_copy(out_v, out_hbm.at[pl.ds(base, SPW)])


@functools.partial(
    pl.kernel,
    out_type=jax.ShapeDtypeStruct((SEGPAD, D), jnp.float32),
    mesh=plsc.VectorSubcoreMesh(core_axis_name="c", subcore_axis_name="s"),
    scratch_types=[
        pltpu.VMEM((SPW, LPAD), jnp.int32),
        pltpu.VMEM((LPAD, D), jnp.float32),
        pltpu.VMEM((LPAD, D), jnp.float32),
        pltpu.VMEM((SPW, D), jnp.float32),
        pltpu.SemaphoreType.DMA,
        pltpu.SemaphoreType.DMA,
    ],
)
def _pool_sc(emb_hbm, idx_hbm, out_hbm, idx_v, rows0, rows1, out_v,
             sem0, sem1):
    _pool_body(emb_hbm, idx_hbm, out_hbm, idx_v, rows0, rows1, out_v,
               sem0, sem1)


def _softmax0(x):
    m = jnp.max(x, axis=0, keepdims=True)
    e = jnp.exp(x - m)
    return e / jnp.sum(e, axis=0, keepdims=True)


def _dense_tc(pooled_ref, rw_ref, out_ref):
    pooled = pooled_ref[...]
    rw = rw_ref[...]
    enc_keys = pooled[0:1000]
    enc_values = pooled[1000:2000]
    enc_cands = pooled[2000:2020]
    enc_persona = pooled[2020:2025]
    enc_x = pooled[2025:2026]

    eps = 1e-6
    dot = jnp.sum(enc_x * enc_persona, axis=1, keepdims=True)
    na = jnp.sqrt(jnp.sum(enc_x * enc_x, axis=1, keepdims=True))
    nb = jnp.sqrt(jnp.sum(enc_persona * enc_persona, axis=1, keepdims=True))
    sim = dot / (jnp.maximum(na, eps) * jnp.maximum(nb, eps))
    ss = _softmax0(sim)
    test = jnp.dot(ss.T, enc_persona, preferred_element_type=jnp.float32)
    q = jnp.dot(test, rw.T, preferred_element_type=jnp.float32)
    tmp = jnp.dot(enc_keys, q.T, preferred_element_type=jnp.float32)
    ph = _softmax0(tmp)
    test2 = jnp.dot(ph.T, enc_values, preferred_element_type=jnp.float32)
    q2 = jnp.dot(test2, rw.T, preferred_element_type=jnp.float32)
    logits = jnp.dot(enc_cands, q2.T, preferred_element_type=jnp.float32)
    out_ref[...] = _softmax0(logits)


def kernel(xs, candidates, persona, label, keys, values, emb_table, R_W):
    del label
    idx = jnp.concatenate([
        keys.reshape(-1), values.reshape(-1), candidates.reshape(-1),
        persona.reshape(-1), xs.reshape(-1),
    ]).astype(jnp.int32).reshape(SEG, L)
    idx_pad = jnp.zeros((SEGPAD, LPAD), jnp.int32).at[:SEG, :L].set(idx)
    pooled = _pool_sc(emb_table.astype(jnp.float32), idx_pad)
    preds = pl.pallas_call(
        _dense_tc,
        out_shape=jax.ShapeDtypeStruct((20, 1), jnp.float32),
    )(pooled, R_W.astype(jnp.float32))
    return preds
